# Initial kernel scaffold; baseline (speedup 1.0000x reference)
import jax
import jax.numpy as jnp
from jax.experimental import pallas as pl


def kernel(phase_table, mag_table, indices):
    raise NotImplementedError("write your pallas kernel here")


# speedup vs baseline: 1.5321x; 1.5321x over previous
import functools

import jax
import jax.numpy as jnp
from jax.experimental import pallas as pl
from jax.experimental.pallas import tpu as pltpu
from jax.experimental.pallas import tpu_sc as plsc

_NUM_CORES = 2
_NUM_SUBCORES = 16
_NW = _NUM_CORES * _NUM_SUBCORES
_CHUNK = 128


def _sc_gather2(phase_table, mag_table, idx3):
    nw, nchunk, chunk = idx3.shape
    batch = nw * nchunk * chunk
    dim = phase_table.shape[1]
    dt = phase_table.dtype
    mesh = plsc.VectorSubcoreMesh(core_axis_name="c", subcore_axis_name="s")

    @functools.partial(
        pl.kernel,
        out_type=(
            jax.ShapeDtypeStruct((batch, dim), dt),
            jax.ShapeDtypeStruct((batch, dim), dt),
        ),
        mesh=mesh,
        scratch_types=[
            pltpu.VMEM((nchunk, chunk), jnp.int32),
            pltpu.VMEM((chunk, dim), dt),
            pltpu.VMEM((chunk, dim), dt),
            pltpu.VMEM((chunk, dim), dt),
            pltpu.VMEM((chunk, dim), dt),
            pltpu.SemaphoreType.DMA,
            pltpu.SemaphoreType.DMA,
            pltpu.SemaphoreType.DMA,
            pltpu.SemaphoreType.DMA,
        ],
    )
    def k(phase_hbm, mag_hbm, idx_hbm, phase_out, mag_out,
          idx_v, p0, p1, m0, m1, sp0, sp1, sm0, sm1):
        wid = jax.lax.axis_index("s") * _NUM_CORES + jax.lax.axis_index("c")
        pltpu.sync_copy(idx_hbm.at[wid], idx_v)
        pbufs, psems = (p0, p1), (sp0, sp1)
        mbufs, msems = (m0, m1), (sm0, sm1)
        base = wid * (nchunk * chunk)

        copies = {
            0: (
                pltpu.async_copy(phase_hbm.at[idx_v.at[0]], pbufs[0], psems[0]),
                pltpu.async_copy(mag_hbm.at[idx_v.at[0]], mbufs[0], msems[0]),
            )
        }
        for j in range(nchunk):
            cur, nxt = j % 2, (j + 1) % 2
            if j + 1 < nchunk:
                copies[j + 1] = (
                    pltpu.async_copy(
                        phase_hbm.at[idx_v.at[j + 1]], pbufs[nxt], psems[nxt]),
                    pltpu.async_copy(
                        mag_hbm.at[idx_v.at[j + 1]], mbufs[nxt], msems[nxt]),
                )
            cp, cm = copies.pop(j)
            out_slc = pl.ds(base + j * chunk, chunk)
            cp.wait()
            pltpu.sync_copy(pbufs[cur], phase_out.at[out_slc])
            cm.wait()
            pltpu.sync_copy(mbufs[cur], mag_out.at[out_slc])

    return k(phase_table, mag_table, idx3)


def kernel(phase_table, mag_table, indices):
    batch = indices.shape[0]
    idx3 = indices.reshape(_NW, batch // (_NW * _CHUNK), _CHUNK)
    phase, mag = _sc_gather2(phase_table, mag_table, idx3)
    return (phase, mag)


# speedup vs baseline: 1.5941x; 1.0405x over previous
import functools

import jax
import jax.numpy as jnp
from jax.experimental import pallas as pl
from jax.experimental.pallas import tpu as pltpu
from jax.experimental.pallas import tpu_sc as plsc

_NUM_CORES = 2
_NUM_SUBCORES = 16
_NW = _NUM_CORES * _NUM_SUBCORES
_CHUNK = 128


def _sc_gather2(phase_table, mag_table, idx3):
    nw, nchunk, chunk = idx3.shape
    batch = nw * nchunk * chunk
    dim = phase_table.shape[1]
    dt = phase_table.dtype
    mesh = plsc.VectorSubcoreMesh(core_axis_name="c", subcore_axis_name="s")

    @functools.partial(
        pl.kernel,
        out_type=(
            jax.ShapeDtypeStruct((batch, dim), dt),
            jax.ShapeDtypeStruct((batch, dim), dt),
        ),
        mesh=mesh,
        scratch_types=(
            [pltpu.VMEM((nchunk, chunk), jnp.int32)]
            + [pltpu.VMEM((chunk, dim), dt) for _ in range(6)]
            + [pltpu.SemaphoreType.DMA for _ in range(12)]
        ),
    )
    def k(phase_hbm, mag_hbm, idx_hbm, phase_out, mag_out, idx_v,
          p0, p1, p2, m0, m1, m2,
          gp0, gp1, gp2, gm0, gm1, gm2, wp0, wp1, wp2, wm0, wm1, wm2):
        wid = jax.lax.axis_index("s") * _NUM_CORES + jax.lax.axis_index("c")
        pltpu.sync_copy(idx_hbm.at[wid], idx_v)
        pbufs, mbufs = (p0, p1, p2), (m0, m1, m2)
        gpsems, gmsems = (gp0, gp1, gp2), (gm0, gm1, gm2)
        wpsems, wmsems = (wp0, wp1, wp2), (wm0, wm1, wm2)
        nbuf = 3
        base = wid * (nchunk * chunk)

        def gather(j):
            s = j % nbuf
            return (
                pltpu.async_copy(phase_hbm.at[idx_v.at[j]], pbufs[s], gpsems[s]),
                pltpu.async_copy(mag_hbm.at[idx_v.at[j]], mbufs[s], gmsems[s]),
            )

        gathers, writes = {}, {}
        for j in range(min(nbuf - 1, nchunk)):
            gathers[j] = gather(j)
        for j in range(nchunk):
            s = j % nbuf
            nj = j + nbuf - 1
            if nj < nchunk:
                ds = nj % nbuf
                for w in writes.pop(nj - nbuf, ()):
                    w.wait()
                gathers[nj] = gather(nj)
            cp, cm = gathers.pop(j)
            out_slc = pl.ds(base + j * chunk, chunk)
            cp.wait()
            writes[j] = [pltpu.async_copy(pbufs[s], phase_out.at[out_slc],
                                          wpsems[s])]
            cm.wait()
            writes[j].append(pltpu.async_copy(mbufs[s], mag_out.at[out_slc],
                                              wmsems[s]))
        for ws in writes.values():
            for w in ws:
                w.wait()

    return k(phase_table, mag_table, idx3)


def kernel(phase_table, mag_table, indices):
    batch = indices.shape[0]
    idx3 = indices.reshape(_NW, batch // (_NW * _CHUNK), _CHUNK)
    phase, mag = _sc_gather2(phase_table, mag_table, idx3)
    return (phase, mag)
